# Initial kernel scaffold; baseline (speedup 1.0000x reference)
import jax
import jax.numpy as jnp
from jax.experimental import pallas as pl


def kernel(input, W):
    raise NotImplementedError("write your pallas kernel here")


# speedup vs baseline: 33.8237x; 33.8237x over previous
import jax
import jax.numpy as jnp
from jax.experimental import pallas as pl

B, S, D_IN, D_OUT = 4, 32, 512, 512
_BIG = 1e30


def _tc_body(x_ref, w_ref, outp_ref, outn_ref):
    a = x_ref[0, 0, :]
    W = w_ref[...]
    ap = jnp.maximum(3.0 + a, 0.0)[:, None]
    an = jnp.maximum(3.0 - a, 0.0)[:, None]
    Wp = jnp.maximum(3.0 + W, 0.0)
    Wn = jnp.maximum(3.0 - W, 0.0)

    def top8mean(A, Bm):
        acc = jnp.zeros((D_OUT,), jnp.float32)
        for _ in range(8):
            m = jnp.minimum(jnp.min(A, axis=0), jnp.min(Bm, axis=0))
            acc = acc + m
            A = jnp.where(A == m[None, :], _BIG, A)
            Bm = jnp.where(Bm == m[None, :], _BIG, Bm)
        return acc * 0.125

    outp_ref[0, 0, :] = top8mean(ap + Wp, an + Wn)
    outn_ref[0, 0, :] = top8mean(ap + Wn, an + Wp)


def kernel(input, W):
    x3 = input.reshape(B * S, 1, D_IN)
    outp, outn = pl.pallas_call(
        _tc_body,
        grid=(B * S,),
        in_specs=[
            pl.BlockSpec((1, 1, D_IN), lambda i: (i, 0, 0)),
            pl.BlockSpec((D_IN, D_OUT), lambda i: (0, 0)),
        ],
        out_specs=[
            pl.BlockSpec((1, 1, D_OUT), lambda i: (i, 0, 0)),
            pl.BlockSpec((1, 1, D_OUT), lambda i: (i, 0, 0)),
        ],
        out_shape=[
            jax.ShapeDtypeStruct((B * S, 1, D_OUT), jnp.float32),
            jax.ShapeDtypeStruct((B * S, 1, D_OUT), jnp.float32),
        ],
    )(x3, W)
    return outp.reshape(B, S, D_OUT), outn.reshape(B, S, D_OUT)


# speedup vs baseline: 68.0043x; 2.0106x over previous
import functools

import jax
import jax.numpy as jnp
from jax import lax
from jax.experimental import pallas as pl
from jax.experimental.pallas import tpu as pltpu
from jax.experimental.pallas import tpu_sc as plsc

B, S, D_IN, D_OUT = 4, 32, 512, 512
N_ROWS = B * S
CAP = 64
BIG = 1e30
L = 16
NC, NS = 2, 16
N_WORKERS = NC * NS
ROWS_PER_W = N_ROWS // N_WORKERS


def _prep_body(w_ref, out_ref):
    out_ref[...] = jnp.full((8, 128), jnp.max(jnp.abs(w_ref[...])), jnp.float32)


def _wabs_max(W):
    return pl.pallas_call(
        _prep_body,
        out_shape=jax.ShapeDtypeStruct((8, 128), jnp.float32),
    )(W)


def _insert8(regs, v):
    out = []
    for r in regs:
        lo = jnp.minimum(r, v)
        v = jnp.maximum(r, v)
        out.append(lo)
    return out


def _shuffle(buf, v, idx):
    buf[...] = v
    return plsc.load_gather(buf, [idx])


def _sort16(buf, v):
    iota = lax.iota(jnp.int32, L)
    for k in (2, 4, 8, 16):
        j = k // 2
        while j >= 1:
            p = _shuffle(buf, v, iota ^ j)
            a_blk = (iota & k) == 0
            lower = (iota & j) == 0
            cond = a_blk == lower
            v = jnp.where(cond, jnp.minimum(v, p), jnp.maximum(v, p))
            j //= 2
    return v


def _prefix_sum16(buf, v):
    iota = lax.iota(jnp.int32, L)
    for d in (1, 2, 4, 8):
        sh = _shuffle(buf, v, jnp.maximum(iota - d, 0))
        v = v + jnp.where(iota >= d, sh, 0)
    return v


def _sc_body(x_hbm, w_hbm, stat_hbm, outp_hbm, outn_hbm,
             xv, statv, sortv, ibuf, alist, slist, idxv, rowsv,
             outpv, outnv, sem):
    cid = lax.axis_index("c")
    sid = lax.axis_index("s")
    wid = sid * NC + cid

    pltpu.sync_copy(stat_hbm, statv)
    spreadv = 2.0 * statv[...]
    idx7 = jnp.full((L,), 7, jnp.int32)
    idx15 = jnp.full((L,), 15, jnp.int32)
    zeros_i = jnp.zeros((L,), jnp.int32)
    bigv = jnp.full((L,), BIG, jnp.float32)
    onev = jnp.full((L,), 1.0, jnp.float32)

    for t in range(ROWS_PER_W):
        row = wid * ROWS_PER_W + t
        pltpu.sync_copy(x_hbm.at[row], xv)

        for k in range(CAP // L):
            idxv[pl.ds(k * L, L)] = zeros_i
            alist[pl.ds(k * L, L)] = bigv
            slist[pl.ds(k * L, L)] = onev

        def passa(j, lmin):
            xj = xv[pl.ds(j * L, L)]
            lmin = jnp.minimum(lmin, jnp.maximum(3.0 + xj, 0.0))
            lmin = jnp.minimum(lmin, jnp.maximum(3.0 - xj, 0.0))
            return lmin

        lmin = lax.fori_loop(0, D_IN // L, passa, bigv)
        ks = _sort16(sortv, lmin)
        ub8 = _shuffle(sortv, ks, idx7)
        tauv = ub8 + spreadv

        def passb(j, offv):
            xj = xv[pl.ds(j * L, L)]
            idx16 = lax.iota(jnp.int32, L) + j * L
            for sgn in (1.0, -1.0):
                aval = jnp.maximum(3.0 + sgn * xj, 0.0)
                msk = aval <= tauv
                mi = msk.astype(jnp.int32)
                cs = _prefix_sum16(ibuf, mi)
                pos = offv + cs - mi
                okm = jnp.logical_and(msk, pos < CAP)
                plsc.store_scatter(alist, [pos], aval, mask=okm)
                plsc.store_scatter(slist, [pos],
                                   jnp.full((L,), sgn, jnp.float32), mask=okm)
                plsc.store_scatter(idxv, [pos], idx16, mask=okm)
                offv = offv + _shuffle(ibuf, cs, idx15)
            return offv

        lax.fori_loop(0, D_IN // L, passb, zeros_i)

        pltpu.async_copy(w_hbm.at[idxv], rowsv, sem).wait()

        def colgroup(cg, _):
            base = cg * L

            def body(r, regs2):
                w = rowsv[r, pl.ds(base, L)]
                rsplat = jnp.broadcast_to(r, (L,)).astype(jnp.int32)
                a16 = plsc.load_gather(alist, [rsplat])
                s16 = plsc.load_gather(slist, [rsplat])
                sw = s16 * w
                cp = a16 + jnp.maximum(3.0 + sw, 0.0)
                cn = a16 + jnp.maximum(3.0 - sw, 0.0)
                return (_insert8(regs2[0], cp), _insert8(regs2[1], cn))

            init = ([bigv] * 8, [bigv] * 8)
            pregs, nregs = lax.fori_loop(0, CAP, body, init)
            sp = pregs[0]
            for r in pregs[1:]:
                sp = sp + r
            sn = nregs[0]
            for r in nregs[1:]:
                sn = sn + r
            outpv[pl.ds(base, L)] = sp * 0.125
            outnv[pl.ds(base, L)] = sn * 0.125
            return 0

        lax.fori_loop(0, D_OUT // L, colgroup, 0)

        pltpu.sync_copy(outpv, outp_hbm.at[row])
        pltpu.sync_copy(outnv, outn_hbm.at[row])


def kernel(input, W):
    x2 = input.reshape(N_ROWS, D_IN)
    stat16 = _wabs_max(W)[0, :L]

    mesh = plsc.VectorSubcoreMesh(core_axis_name="c", subcore_axis_name="s",
                                  num_cores=NC)
    sc = functools.partial(
        pl.kernel,
        mesh=mesh,
        compiler_params=pltpu.CompilerParams(needs_layout_passes=False),
        out_type=[
            jax.ShapeDtypeStruct((N_ROWS, D_OUT), jnp.float32),
            jax.ShapeDtypeStruct((N_ROWS, D_OUT), jnp.float32),
        ],
        scratch_types=[
            pltpu.VMEM((D_IN,), jnp.float32),
            pltpu.VMEM((L,), jnp.float32),
            pltpu.VMEM((L,), jnp.float32),
            pltpu.VMEM((L,), jnp.int32),
            pltpu.VMEM((CAP,), jnp.float32),
            pltpu.VMEM((CAP,), jnp.float32),
            pltpu.VMEM((CAP,), jnp.int32),
            pltpu.VMEM((CAP, D_OUT), jnp.float32),
            pltpu.VMEM((D_OUT,), jnp.float32),
            pltpu.VMEM((D_OUT,), jnp.float32),
            pltpu.SemaphoreType.DMA,
        ],
    )(_sc_body)
    outp, outn = sc(x2, W, stat16)
    return outp.reshape(B, S, D_OUT), outn.reshape(B, S, D_OUT)


# speedup vs baseline: 72.6246x; 1.0679x over previous
import functools

import jax
import jax.numpy as jnp
from jax import lax
from jax.experimental import pallas as pl
from jax.experimental.pallas import tpu as pltpu
from jax.experimental.pallas import tpu_sc as plsc

B, S, D_IN, D_OUT = 4, 32, 512, 512
N_ROWS = B * S
CAP = 128
BIG = 1e30
L = 16
NC, NS = 2, 16
N_WORKERS = NC * NS
ROWS_PER_W = N_ROWS // N_WORKERS


def _prep_body(w_ref, out_ref):
    out_ref[...] = jnp.full((8, 128), jnp.max(jnp.abs(w_ref[...])), jnp.float32)


def _wabs_max(W):
    return pl.pallas_call(
        _prep_body,
        out_shape=jax.ShapeDtypeStruct((8, 128), jnp.float32),
    )(W)


def _insert8(regs, v):
    out = []
    for r in regs:
        lo = jnp.minimum(r, v)
        v = jnp.maximum(r, v)
        out.append(lo)
    return out


def _shuffle(buf, v, idx):
    buf[...] = v
    return plsc.load_gather(buf, [idx])


def _sort16(buf, v):
    iota = lax.iota(jnp.int32, L)
    for k in (2, 4, 8, 16):
        j = k // 2
        while j >= 1:
            p = _shuffle(buf, v, iota ^ j)
            a_blk = (iota & k) == 0
            lower = (iota & j) == 0
            cond = a_blk == lower
            v = jnp.where(cond, jnp.minimum(v, p), jnp.maximum(v, p))
            j //= 2
    return v


def _prefix_sum16(buf, v):
    iota = lax.iota(jnp.int32, L)
    for d in (1, 2, 4, 8):
        sh = _shuffle(buf, v, jnp.maximum(iota - d, 0))
        v = v + jnp.where(iota >= d, sh, 0)
    return v


def _sc_body(x_hbm, w_hbm, stat_hbm, outp_hbm, outn_hbm,
             xv, statv, sortv, ibuf, alist, slist, idxv, rowsv,
             outpv, outnv, sem):
    cid = lax.axis_index("c")
    sid = lax.axis_index("s")
    wid = sid * NC + cid

    pltpu.sync_copy(stat_hbm, statv)
    spreadv = 2.0 * statv[...]
    idx7 = jnp.full((L,), 7, jnp.int32)
    idx15 = jnp.full((L,), 15, jnp.int32)
    zeros_i = jnp.zeros((L,), jnp.int32)
    bigv = jnp.full((L,), BIG, jnp.float32)
    onev = jnp.full((L,), 1.0, jnp.float32)

    for t in range(ROWS_PER_W):
        row = wid * ROWS_PER_W + t
        pltpu.sync_copy(x_hbm.at[row], xv)

        for k in range(CAP // L):
            idxv[pl.ds(k * L, L)] = zeros_i
            alist[pl.ds(k * L, L)] = bigv
            slist[pl.ds(k * L, L)] = onev

        def passa(j, lmin):
            xj = xv[pl.ds(j * L, L)]
            lmin = jnp.minimum(lmin, jnp.maximum(3.0 + xj, 0.0))
            lmin = jnp.minimum(lmin, jnp.maximum(3.0 - xj, 0.0))
            return lmin

        lmin = lax.fori_loop(0, D_IN // L, passa, bigv)
        ks = _sort16(sortv, lmin)
        ub8 = _shuffle(sortv, ks, idx7)
        tauv = ub8 + spreadv

        def passb(j, offv):
            xj = xv[pl.ds(j * L, L)]
            idx16 = lax.iota(jnp.int32, L) + j * L
            for sgn in (1.0, -1.0):
                aval = jnp.maximum(3.0 + sgn * xj, 0.0)
                msk = aval <= tauv
                mi = msk.astype(jnp.int32)
                cs = _prefix_sum16(ibuf, mi)
                pos = offv + cs - mi
                okm = jnp.logical_and(msk, pos < CAP)
                plsc.store_scatter(alist, [pos], aval, mask=okm)
                plsc.store_scatter(slist, [pos],
                                   jnp.full((L,), sgn, jnp.float32), mask=okm)
                plsc.store_scatter(idxv, [pos], idx16, mask=okm)
                offv = offv + _shuffle(ibuf, cs, idx15)
            return offv

        offv = lax.fori_loop(0, D_IN // L, passb, zeros_i)
        n_eff = jnp.minimum(offv[0], CAP)

        half = CAP // 2
        pltpu.async_copy(w_hbm.at[idxv.at[pl.ds(0, half)]],
                         rowsv.at[pl.ds(0, half)], sem).wait()

        @pl.when(n_eff > half)
        def _():
            pltpu.async_copy(w_hbm.at[idxv.at[pl.ds(half, half)]],
                             rowsv.at[pl.ds(half, half)], sem).wait()

        def colgroup(cg, _):
            base0 = cg * (2 * L)
            base1 = base0 + L

            def body(r, regs4):
                rsplat = jnp.broadcast_to(r, (L,)).astype(jnp.int32)
                a16 = plsc.load_gather(alist, [rsplat])
                s16 = plsc.load_gather(slist, [rsplat])
                w0 = rowsv[r, pl.ds(base0, L)]
                w1 = rowsv[r, pl.ds(base1, L)]
                sw0 = s16 * w0
                sw1 = s16 * w1
                cp0 = a16 + jnp.maximum(3.0 + sw0, 0.0)
                cn0 = a16 + jnp.maximum(3.0 - sw0, 0.0)
                cp1 = a16 + jnp.maximum(3.0 + sw1, 0.0)
                cn1 = a16 + jnp.maximum(3.0 - sw1, 0.0)
                return (_insert8(regs4[0], cp0), _insert8(regs4[1], cn0),
                        _insert8(regs4[2], cp1), _insert8(regs4[3], cn1))

            init = ([bigv] * 8, [bigv] * 8, [bigv] * 8, [bigv] * 8)
            p0, n0r, p1, n1r = lax.fori_loop(0, n_eff, body, init)

            def _sum8(regs):
                s = regs[0]
                for r in regs[1:]:
                    s = s + r
                return s * 0.125

            outpv[pl.ds(base0, L)] = _sum8(p0)
            outnv[pl.ds(base0, L)] = _sum8(n0r)
            outpv[pl.ds(base1, L)] = _sum8(p1)
            outnv[pl.ds(base1, L)] = _sum8(n1r)
            return 0

        lax.fori_loop(0, D_OUT // (2 * L), colgroup, 0)

        pltpu.sync_copy(outpv, outp_hbm.at[row])
        pltpu.sync_copy(outnv, outn_hbm.at[row])


def kernel(input, W):
    x2 = input.reshape(N_ROWS, D_IN)
    stat16 = _wabs_max(W)[0, :L]

    mesh = plsc.VectorSubcoreMesh(core_axis_name="c", subcore_axis_name="s",
                                  num_cores=NC)
    sc = functools.partial(
        pl.kernel,
        mesh=mesh,
        compiler_params=pltpu.CompilerParams(needs_layout_passes=False),
        out_type=[
            jax.ShapeDtypeStruct((N_ROWS, D_OUT), jnp.float32),
            jax.ShapeDtypeStruct((N_ROWS, D_OUT), jnp.float32),
        ],
        scratch_types=[
            pltpu.VMEM((D_IN,), jnp.float32),
            pltpu.VMEM((L,), jnp.float32),
            pltpu.VMEM((L,), jnp.float32),
            pltpu.VMEM((L,), jnp.int32),
            pltpu.VMEM((CAP,), jnp.float32),
            pltpu.VMEM((CAP,), jnp.float32),
            pltpu.VMEM((CAP,), jnp.int32),
            pltpu.VMEM((CAP, D_OUT), jnp.float32),
            pltpu.VMEM((D_OUT,), jnp.float32),
            pltpu.VMEM((D_OUT,), jnp.float32),
            pltpu.SemaphoreType.DMA,
        ],
    )(_sc_body)
    outp, outn = sc(x2, W, stat16)
    return outp.reshape(B, S, D_OUT), outn.reshape(B, S, D_OUT)
